# baseline (speedup 1.0000x reference)
import functools

import jax
import jax.numpy as jnp
from jax import lax
from jax.experimental import pallas as pl
from jax.experimental.pallas import tpu as pltpu
from jax.experimental.pallas import tpu_sc as plsc

_D = 64
_PK = 128
_L = 16
_NC = 2
_NS = 16
_NW = _NC * _NS
_IDX_CHUNK = 128
_BR = 4096


def _pack_node(table_t):
    n = table_t.shape[1]

    def body(x_ref, i_ref, o_ref):
        o_ref[:, 0:_D] = jax.lax.dot_general(
            x_ref[...], i_ref[...], (((0,), (0,)), ((), ())),
            preferred_element_type=jnp.float32,
            precision=jax.lax.Precision.HIGHEST)

    return pl.pallas_call(
        body,
        grid=(pl.cdiv(n, _BR),),
        in_specs=[pl.BlockSpec((_D, _BR), lambda b: (0, b)),
                  pl.BlockSpec((_D, _D), lambda b: (0, 0))],
        out_specs=pl.BlockSpec((_BR, _PK), lambda b: (b, 0)),
        out_shape=jax.ShapeDtypeStruct((n, _PK), jnp.float32),
    )(table_t, jnp.eye(_D, dtype=jnp.float32))


def _pack_path(table_t):
    p = table_t.shape[1]

    def body(x_ref, o_ref):
        o_ref[:, 0:_D] = (x_ref[...] > 0.0).astype(jnp.float32).T

    return pl.pallas_call(
        body,
        in_specs=[pl.BlockSpec((_D, p), lambda: (0, 0))],
        out_specs=pl.BlockSpec((p, _PK), lambda: (0, 0)),
        out_shape=jax.ShapeDtypeStruct((p, _PK), jnp.float32),
    )(table_t)


@functools.lru_cache(maxsize=None)
def _build_sc(batch: int):
    bpw = batch // _NW
    n_chunks = bpw // _IDX_CHUNK
    mesh = plsc.VectorSubcoreMesh(core_axis_name="c", subcore_axis_name="s")

    @functools.partial(
        pl.kernel,
        mesh=mesh,
        out_type=jax.ShapeDtypeStruct((batch,), jnp.float32),
        compiler_params=pltpu.CompilerParams(
            needs_layout_passes=False, use_tc_tiling_on_sc=True),
        scratch_types=[
            pltpu.VMEM((bpw,), jnp.int32),
            pltpu.VMEM((bpw,), jnp.int32),
            pltpu.VMEM((bpw,), jnp.int32),
            pltpu.VMEM((2, _IDX_CHUNK, _PK), jnp.float32),
            pltpu.VMEM((2, _IDX_CHUNK, _PK), jnp.float32),
            pltpu.VMEM((bpw,), jnp.float32),
            pltpu.VMEM((128, _PK), jnp.float32),
            pltpu.SemaphoreType.DMA,
            pltpu.SemaphoreType.DMA,
        ],
    )
    def hin2vec_sc(sidx_hbm, eidx_hbm, pidx_hbm, ntab_hbm, ptab_hbm, out_hbm,
                   sidx_v, eidx_v, pidx_v, srow_v, erow_v, out_v, ptab_v,
                   sem0, sem1):
        wid = lax.axis_index("s") * _NC + lax.axis_index("c")
        base = wid * bpw
        sems = (sem0, sem1)

        pltpu.sync_copy(sidx_hbm.at[pl.ds(base, bpw)], sidx_v)
        pltpu.sync_copy(eidx_hbm.at[pl.ds(base, bpw)], eidx_v)
        pltpu.sync_copy(pidx_hbm.at[pl.ds(base, bpw)], pidx_v)
        pltpu.sync_copy(ptab_hbm, ptab_v.at[pl.ds(0, 100)])

        lane_ids = lax.iota(jnp.int32, _L)

        def fire(ch):
            b = ch % 2
            sl = pl.ds(ch * _IDX_CHUNK, _IDX_CHUNK)
            return (
                pltpu.async_copy(
                    ntab_hbm.at[sidx_v.at[sl]], srow_v.at[b], sems[b]),
                pltpu.async_copy(
                    ntab_hbm.at[eidx_v.at[sl]], erow_v.at[b], sems[b]),
            )

        pending = {0: fire(0)}
        for ch in range(n_chunks):
            b = ch % 2
            if ch + 1 < n_chunks:
                pending[ch + 1] = fire(ch + 1)
            for cp in pending.pop(ch):
                cp.wait()
            sbuf = srow_v.at[b]
            ebuf = erow_v.at[b]

            def group(g, _, sbuf=sbuf, ebuf=ebuf, ch=ch):
                gbase = g * _L
                pidx_vec = pidx_v[pl.ds(ch * _IDX_CHUNK + gbase, _L)]
                w = jnp.zeros((_L,), jnp.float32)
                for i in range(_L):
                    prow = pidx_vec[i]
                    acc = jnp.zeros((_L,), jnp.float32)
                    for c in range(_D // _L):
                        dsl = pl.ds(c * _L, _L)
                        sv = sbuf[gbase + i, dsl]
                        ev = ebuf[gbase + i, dsl]
                        pv = ptab_v[prow, dsl]
                        acc = acc + sv * ev * pv
                    w = jnp.where(lane_ids == i, jnp.sum(acc), w)
                out_v[pl.ds(ch * _IDX_CHUNK + gbase, _L)] = (
                    1.0 / (1.0 + jnp.exp(-w)))
                return _

            lax.fori_loop(0, _IDX_CHUNK // _L, group, 0)

        pltpu.sync_copy(out_v, out_hbm.at[pl.ds(base, bpw)])

    return hin2vec_sc


def kernel(start_node, end_node, path, node_table, path_table):
    packed = _pack_node(node_table.T)
    ppacked = _pack_path(path_table.T)
    fn = _build_sc(start_node.shape[0])
    return fn(start_node.astype(jnp.int32), end_node.astype(jnp.int32),
              path.astype(jnp.int32), packed, ppacked)


# speedup vs baseline: 1.6883x; 1.6883x over previous
import functools

import jax
import jax.numpy as jnp
from jax import lax
from jax.experimental import pallas as pl
from jax.experimental.pallas import tpu as pltpu
from jax.experimental.pallas import tpu_sc as plsc

_D = 64
_PK = 128
_L = 16
_NC = 2
_NS = 16
_NW = _NC * _NS
_IDX_CHUNK = 128
_BR = 4096


_NPACK = 524288
_SPLIT = 479232


def _pack_node(table_t):
    def body(x1_ref, x2_ref, o_ref):
        o_ref[:, 0:_D] = x1_ref[...].T
        o_ref[:, _D:_PK] = x2_ref[...].T

    return pl.pallas_call(
        body,
        grid=(_NPACK // _BR,),
        in_specs=[pl.BlockSpec((_D, _BR), lambda b: (0, b)),
                  pl.BlockSpec((_D, _BR), lambda b: (0, b + _SPLIT // _BR))],
        out_specs=pl.BlockSpec((_BR, _PK), lambda b: (b, 0)),
        out_shape=jax.ShapeDtypeStruct((_NPACK, _PK), jnp.float32),
    )(table_t, table_t)


def _pack_path(table_t):
    p = table_t.shape[1]

    def body(x_ref, o_ref):
        o_ref[:, 0:_D] = (x_ref[...] > 0.0).astype(jnp.float32).T

    return pl.pallas_call(
        body,
        in_specs=[pl.BlockSpec((_D, p), lambda: (0, 0))],
        out_specs=pl.BlockSpec((p, _PK), lambda: (0, 0)),
        out_shape=jax.ShapeDtypeStruct((p, _PK), jnp.float32),
    )(table_t)


@functools.lru_cache(maxsize=None)
def _build_sc(batch: int):
    bpw = batch // _NW
    n_chunks = bpw // _IDX_CHUNK
    mesh = plsc.VectorSubcoreMesh(core_axis_name="c", subcore_axis_name="s")

    @functools.partial(
        pl.kernel,
        mesh=mesh,
        out_type=jax.ShapeDtypeStruct((batch,), jnp.float32),
        compiler_params=pltpu.CompilerParams(
            needs_layout_passes=False, use_tc_tiling_on_sc=True),
        scratch_types=[
            pltpu.VMEM((bpw,), jnp.int32),
            pltpu.VMEM((bpw,), jnp.int32),
            pltpu.VMEM((bpw,), jnp.int32),
            pltpu.VMEM((bpw,), jnp.int32),
            pltpu.VMEM((bpw,), jnp.int32),
            pltpu.VMEM((2, _IDX_CHUNK, _PK), jnp.float32),
            pltpu.VMEM((2, _IDX_CHUNK, _PK), jnp.float32),
            pltpu.VMEM((bpw,), jnp.float32),
            pltpu.VMEM((128, _PK), jnp.float32),
            pltpu.SemaphoreType.DMA,
            pltpu.SemaphoreType.DMA,
        ],
    )
    def hin2vec_sc(sidx_hbm, eidx_hbm, pidx_hbm, ntab_hbm, ptab_hbm, out_hbm,
                   sidx_v, eidx_v, pidx_v, scol_v, ecol_v, srow_v, erow_v,
                   out_v, ptab_v, sem0, sem1):
        wid = lax.axis_index("s") * _NC + lax.axis_index("c")
        base = wid * bpw
        sems = (sem0, sem1)

        pltpu.sync_copy(sidx_hbm.at[pl.ds(base, bpw)], sidx_v)
        pltpu.sync_copy(eidx_hbm.at[pl.ds(base, bpw)], eidx_v)
        pltpu.sync_copy(pidx_hbm.at[pl.ds(base, bpw)], pidx_v)
        pltpu.sync_copy(ptab_hbm, ptab_v.at[pl.ds(0, 100)])

        def remap(j, _):
            vsl = pl.ds(j * _L, _L)
            for idx_v, col_v in ((sidx_v, scol_v), (eidx_v, ecol_v)):
                v = idx_v[vsl]
                hi = v >= _SPLIT
                idx_v[vsl] = jnp.where(hi, v - _SPLIT, v)
                col_v[vsl] = jnp.where(hi, _D, 0)
            return _

        lax.fori_loop(0, bpw // _L, remap, 0)

        lane_ids = lax.iota(jnp.int32, _L)

        def fire(ch):
            b = ch % 2
            sl = pl.ds(ch * _IDX_CHUNK, _IDX_CHUNK)
            return (
                pltpu.async_copy(
                    ntab_hbm.at[sidx_v.at[sl]], srow_v.at[b], sems[b]),
                pltpu.async_copy(
                    ntab_hbm.at[eidx_v.at[sl]], erow_v.at[b], sems[b]),
            )

        pending = {0: fire(0)}
        for ch in range(n_chunks):
            b = ch % 2
            if ch + 1 < n_chunks:
                pending[ch + 1] = fire(ch + 1)
            for cp in pending.pop(ch):
                cp.wait()
            sbuf = srow_v.at[b]
            ebuf = erow_v.at[b]

            def group(g, _, sbuf=sbuf, ebuf=ebuf, ch=ch):
                gbase = g * _L
                gsl = pl.ds(ch * _IDX_CHUNK + gbase, _L)
                pidx_vec = pidx_v[gsl]
                scol_vec = scol_v[gsl]
                ecol_vec = ecol_v[gsl]
                w = jnp.zeros((_L,), jnp.float32)
                for i in range(_L):
                    prow = pidx_vec[i]
                    scol = scol_vec[i]
                    ecol = ecol_vec[i]
                    acc = jnp.zeros((_L,), jnp.float32)
                    for c in range(_D // _L):
                        dsl = pl.ds(c * _L, _L)
                        sv = sbuf[gbase + i, pl.ds(scol + c * _L, _L)]
                        ev = ebuf[gbase + i, pl.ds(ecol + c * _L, _L)]
                        pv = ptab_v[prow, dsl]
                        acc = acc + sv * ev * pv
                    w = jnp.where(lane_ids == i, jnp.sum(acc), w)
                out_v[gsl] = 1.0 / (1.0 + jnp.exp(-w))
                return _

            lax.fori_loop(0, _IDX_CHUNK // _L, group, 0)

        pltpu.sync_copy(out_v, out_hbm.at[pl.ds(base, bpw)])

    return hin2vec_sc


def kernel(start_node, end_node, path, node_table, path_table):
    packed = _pack_node(node_table.T)
    ppacked = _pack_path(path_table.T)
    fn = _build_sc(start_node.shape[0])
    return fn(start_node.astype(jnp.int32), end_node.astype(jnp.int32),
              path.astype(jnp.int32), packed, ppacked)


# speedup vs baseline: 1.9131x; 1.1331x over previous
import functools

import jax
import jax.numpy as jnp
from jax import lax
from jax.experimental import pallas as pl
from jax.experimental.pallas import tpu as pltpu
from jax.experimental.pallas import tpu_sc as plsc

_D = 64
_PK = 128
_L = 16
_NC = 2
_NS = 16
_NW = _NC * _NS
_IDX_CHUNK = 128
_BR = 8192


_NPACK = 524288
_SPLIT = 483328


def _pack_node(table_t):
    def body(x1_ref, x2_ref, o_ref):
        o_ref[:, 0:_D] = x1_ref[...].T
        o_ref[:, _D:_PK] = x2_ref[...].T

    return pl.pallas_call(
        body,
        grid=(_NPACK // _BR,),
        in_specs=[pl.BlockSpec((_D, _BR), lambda b: (0, b)),
                  pl.BlockSpec((_D, _BR), lambda b: (0, b + _SPLIT // _BR))],
        out_specs=pl.BlockSpec((_BR, _PK), lambda b: (b, 0)),
        out_shape=jax.ShapeDtypeStruct((_NPACK, _PK), jnp.float32),
    )(table_t, table_t)


def _pack_path(table_t):
    p = table_t.shape[1]

    def body(x_ref, o_ref):
        o_ref[:, 0:_D] = (x_ref[...] > 0.0).astype(jnp.float32).T

    return pl.pallas_call(
        body,
        in_specs=[pl.BlockSpec((_D, p), lambda: (0, 0))],
        out_specs=pl.BlockSpec((p, _PK), lambda: (0, 0)),
        out_shape=jax.ShapeDtypeStruct((p, _PK), jnp.float32),
    )(table_t)


@functools.lru_cache(maxsize=None)
def _build_sc(batch: int):
    bpw = batch // _NW
    n_chunks = bpw // _IDX_CHUNK
    mesh = plsc.VectorSubcoreMesh(core_axis_name="c", subcore_axis_name="s")

    @functools.partial(
        pl.kernel,
        mesh=mesh,
        out_type=jax.ShapeDtypeStruct((batch,), jnp.float32),
        compiler_params=pltpu.CompilerParams(
            needs_layout_passes=False, use_tc_tiling_on_sc=True),
        scratch_types=[
            pltpu.VMEM((bpw,), jnp.int32),
            pltpu.VMEM((bpw,), jnp.int32),
            pltpu.VMEM((bpw,), jnp.int32),
            pltpu.VMEM((bpw,), jnp.int32),
            pltpu.VMEM((bpw,), jnp.int32),
            pltpu.VMEM((2, _IDX_CHUNK, _PK), jnp.float32),
            pltpu.VMEM((2, _IDX_CHUNK, _PK), jnp.float32),
            pltpu.VMEM((bpw,), jnp.float32),
            pltpu.VMEM((128, _PK), jnp.float32),
            pltpu.SemaphoreType.DMA,
            pltpu.SemaphoreType.DMA,
        ],
    )
    def hin2vec_sc(sidx_hbm, eidx_hbm, pidx_hbm, ntab_hbm, ptab_hbm, out_hbm,
                   sidx_v, eidx_v, pidx_v, scol_v, ecol_v, srow_v, erow_v,
                   out_v, ptab_v, sem0, sem1):
        wid = lax.axis_index("s") * _NC + lax.axis_index("c")
        base = wid * bpw
        sems = (sem0, sem1)

        pltpu.sync_copy(sidx_hbm.at[pl.ds(base, bpw)], sidx_v)
        pltpu.sync_copy(eidx_hbm.at[pl.ds(base, bpw)], eidx_v)
        pltpu.sync_copy(pidx_hbm.at[pl.ds(base, bpw)], pidx_v)
        pltpu.sync_copy(ptab_hbm, ptab_v.at[pl.ds(0, 100)])

        def remap(j, _):
            vsl = pl.ds(j * _L, _L)
            for idx_v, col_v in ((sidx_v, scol_v), (eidx_v, ecol_v)):
                v = idx_v[vsl]
                hi = v >= _SPLIT
                idx_v[vsl] = jnp.where(hi, v - _SPLIT, v)
                col_v[vsl] = jnp.where(hi, _D, 0)
            return _

        lax.fori_loop(0, bpw // _L, remap, 0)

        lane_ids = lax.iota(jnp.int32, _L)

        def fire(ch):
            b = ch % 2
            sl = pl.ds(ch * _IDX_CHUNK, _IDX_CHUNK)
            return (
                pltpu.async_copy(
                    ntab_hbm.at[sidx_v.at[sl]], srow_v.at[b], sems[b]),
                pltpu.async_copy(
                    ntab_hbm.at[eidx_v.at[sl]], erow_v.at[b], sems[b]),
            )

        pending = {0: fire(0)}
        for ch in range(n_chunks):
            b = ch % 2
            if ch + 1 < n_chunks:
                pending[ch + 1] = fire(ch + 1)
            for cp in pending.pop(ch):
                cp.wait()
            sbuf = srow_v.at[b]
            ebuf = erow_v.at[b]

            def group(g, _, sbuf=sbuf, ebuf=ebuf, ch=ch):
                gbase = g * _L
                gsl = pl.ds(ch * _IDX_CHUNK + gbase, _L)
                pidx_vec = pidx_v[gsl]
                scol_vec = scol_v[gsl]
                ecol_vec = ecol_v[gsl]
                w = jnp.zeros((_L,), jnp.float32)
                for i in range(_L):
                    prow = pidx_vec[i]
                    scol = scol_vec[i]
                    ecol = ecol_vec[i]
                    acc = jnp.zeros((_L,), jnp.float32)
                    for c in range(_D // _L):
                        dsl = pl.ds(c * _L, _L)
                        sv = sbuf[gbase + i, pl.ds(scol + c * _L, _L)]
                        ev = ebuf[gbase + i, pl.ds(ecol + c * _L, _L)]
                        pv = ptab_v[prow, dsl]
                        acc = acc + sv * ev * pv
                    w = jnp.where(lane_ids == i, jnp.sum(acc), w)
                out_v[gsl] = 1.0 / (1.0 + jnp.exp(-w))
                return _

            lax.fori_loop(0, _IDX_CHUNK // _L, group, 0)

        pltpu.sync_copy(out_v, out_hbm.at[pl.ds(base, bpw)])

    return hin2vec_sc


def kernel(start_node, end_node, path, node_table, path_table):
    packed = _pack_node(node_table.T)
    ppacked = _pack_path(path_table.T)
    fn = _build_sc(start_node.shape[0])
    return fn(start_node.astype(jnp.int32), end_node.astype(jnp.int32),
              path.astype(jnp.int32), packed, ppacked)


# speedup vs baseline: 2.0216x; 1.0567x over previous
import functools

import jax
import jax.numpy as jnp
from jax import lax
from jax.experimental import pallas as pl
from jax.experimental.pallas import tpu as pltpu
from jax.experimental.pallas import tpu_sc as plsc

_D = 64
_PK = 128
_L = 16
_NC = 2
_NS = 16
_NW = _NC * _NS
_IDX_CHUNK = 128
_BR = 16384


_NPACK = 524288
_SPLIT = 491520


def _pack_node(table_t):
    def body(x1_ref, x2_ref, o_ref):
        o_ref[:, 0:_D] = x1_ref[...].T
        o_ref[:, _D:_PK] = x2_ref[...].T

    return pl.pallas_call(
        body,
        grid=(_NPACK // _BR,),
        in_specs=[pl.BlockSpec((_D, _BR), lambda b: (0, b)),
                  pl.BlockSpec((_D, _BR), lambda b: (0, b + _SPLIT // _BR))],
        out_specs=pl.BlockSpec((_BR, _PK), lambda b: (b, 0)),
        out_shape=jax.ShapeDtypeStruct((_NPACK, _PK), jnp.float32),
    )(table_t, table_t)


def _pack_path(table_t):
    p = table_t.shape[1]

    def body(x_ref, o_ref):
        o_ref[:, 0:_D] = (x_ref[...] > 0.0).astype(jnp.float32).T

    return pl.pallas_call(
        body,
        in_specs=[pl.BlockSpec((_D, p), lambda: (0, 0))],
        out_specs=pl.BlockSpec((p, _PK), lambda: (0, 0)),
        out_shape=jax.ShapeDtypeStruct((p, _PK), jnp.float32),
    )(table_t)


@functools.lru_cache(maxsize=None)
def _build_sc(batch: int):
    bpw = batch // _NW
    n_chunks = bpw // _IDX_CHUNK
    mesh = plsc.VectorSubcoreMesh(core_axis_name="c", subcore_axis_name="s")

    @functools.partial(
        pl.kernel,
        mesh=mesh,
        out_type=jax.ShapeDtypeStruct((batch,), jnp.float32),
        compiler_params=pltpu.CompilerParams(
            needs_layout_passes=False, use_tc_tiling_on_sc=True),
        scratch_types=[
            pltpu.VMEM((bpw,), jnp.int32),
            pltpu.VMEM((bpw,), jnp.int32),
            pltpu.VMEM((bpw,), jnp.int32),
            pltpu.VMEM((bpw,), jnp.int32),
            pltpu.VMEM((bpw,), jnp.int32),
            pltpu.VMEM((2, _IDX_CHUNK, _PK), jnp.float32),
            pltpu.VMEM((2, _IDX_CHUNK, _PK), jnp.float32),
            pltpu.VMEM((bpw,), jnp.float32),
            pltpu.VMEM((128, _PK), jnp.float32),
            pltpu.SemaphoreType.DMA,
            pltpu.SemaphoreType.DMA,
        ],
    )
    def hin2vec_sc(sidx_hbm, eidx_hbm, pidx_hbm, ntab_hbm, ptab_hbm, out_hbm,
                   sidx_v, eidx_v, pidx_v, scol_v, ecol_v, srow_v, erow_v,
                   out_v, ptab_v, sem0, sem1):
        wid = lax.axis_index("s") * _NC + lax.axis_index("c")
        base = wid * bpw
        sems = (sem0, sem1)

        pltpu.sync_copy(sidx_hbm.at[pl.ds(base, bpw)], sidx_v)
        pltpu.sync_copy(eidx_hbm.at[pl.ds(base, bpw)], eidx_v)
        pltpu.sync_copy(pidx_hbm.at[pl.ds(base, bpw)], pidx_v)
        pltpu.sync_copy(ptab_hbm, ptab_v.at[pl.ds(0, 100)])

        def remap(j, _):
            vsl = pl.ds(j * _L, _L)
            for idx_v, col_v in ((sidx_v, scol_v), (eidx_v, ecol_v)):
                v = idx_v[vsl]
                hi = v >= _SPLIT
                idx_v[vsl] = jnp.where(hi, v - _SPLIT, v)
                col_v[vsl] = jnp.where(hi, _D, 0)
            return _

        lax.fori_loop(0, bpw // _L, remap, 0)

        lane_ids = lax.iota(jnp.int32, _L)

        def fire(ch):
            b = ch % 2
            sl = pl.ds(ch * _IDX_CHUNK, _IDX_CHUNK)
            return (
                pltpu.async_copy(
                    ntab_hbm.at[sidx_v.at[sl]], srow_v.at[b], sems[b]),
                pltpu.async_copy(
                    ntab_hbm.at[eidx_v.at[sl]], erow_v.at[b], sems[b]),
            )

        pending = {0: fire(0)}
        for ch in range(n_chunks):
            b = ch % 2
            if ch + 1 < n_chunks:
                pending[ch + 1] = fire(ch + 1)
            for cp in pending.pop(ch):
                cp.wait()
            sbuf = srow_v.at[b]
            ebuf = erow_v.at[b]

            def group(g, _, sbuf=sbuf, ebuf=ebuf, ch=ch):
                gbase = g * _L
                gsl = pl.ds(ch * _IDX_CHUNK + gbase, _L)
                pidx_vec = pidx_v[gsl]
                scol_vec = scol_v[gsl]
                ecol_vec = ecol_v[gsl]
                w = jnp.zeros((_L,), jnp.float32)
                for i in range(_L):
                    prow = pidx_vec[i]
                    scol = scol_vec[i]
                    ecol = ecol_vec[i]
                    acc = jnp.zeros((_L,), jnp.float32)
                    for c in range(_D // _L):
                        dsl = pl.ds(c * _L, _L)
                        sv = sbuf[gbase + i, pl.ds(scol + c * _L, _L)]
                        ev = ebuf[gbase + i, pl.ds(ecol + c * _L, _L)]
                        pv = ptab_v[prow, dsl]
                        acc = acc + sv * ev * pv
                    w = jnp.where(lane_ids == i, jnp.sum(acc), w)
                out_v[gsl] = 1.0 / (1.0 + jnp.exp(-w))
                return _

            lax.fori_loop(0, _IDX_CHUNK // _L, group, 0)

        pltpu.sync_copy(out_v, out_hbm.at[pl.ds(base, bpw)])

    return hin2vec_sc


def kernel(start_node, end_node, path, node_table, path_table):
    packed = _pack_node(node_table.T)
    ppacked = _pack_path(path_table.T)
    fn = _build_sc(start_node.shape[0])
    return fn(start_node.astype(jnp.int32), end_node.astype(jnp.int32),
              path.astype(jnp.int32), packed, ppacked)
